# Initial kernel scaffold; baseline (speedup 1.0000x reference)
import jax
import jax.numpy as jnp
from jax.experimental import pallas as pl


def kernel(indices_item, indices_cate, item_table, cate_table):
    raise NotImplementedError("write your pallas kernel here")


# speedup vs baseline: 4.8139x; 4.8139x over previous
import functools

import jax
import jax.numpy as jnp
from jax import lax
from jax.experimental import pallas as pl
from jax.experimental.pallas import tpu as pltpu
from jax.experimental.pallas import tpu_sc as plsc

BATCH = 4096
HIST = 50
DIM = 128
TOTAL = BATCH * HIST

_info = plsc.get_sparse_core_info()
_NC, _NS = _info.num_cores, _info.num_subcores
_NW = _NC * _NS
_PER_W = TOTAL // _NW
_CHUNK = 128
_NCHUNK = _PER_W // _CHUNK

_mesh = plsc.VectorSubcoreMesh(core_axis_name="c", subcore_axis_name="s")


@functools.partial(
    pl.kernel,
    mesh=_mesh,
    out_type=(
        jax.ShapeDtypeStruct((TOTAL, DIM), jnp.float32),
        jax.ShapeDtypeStruct((TOTAL, DIM), jnp.float32),
    ),
    scratch_types=[
        pltpu.VMEM((_CHUNK,), jnp.int32),
        pltpu.VMEM((_CHUNK, DIM), jnp.float32),
        pltpu.SemaphoreType.DMA,
    ],
)
def _gather_kernel(idx_item_hbm, idx_cate_hbm, item_tab_hbm, cate_tab_hbm,
                   out_item_hbm, out_cate_hbm, idx_v, rows_v, sem):
    wid = lax.axis_index("s") * _NC + lax.axis_index("c")
    base = wid * _PER_W

    def do_table(idx_hbm, tab_hbm, out_hbm):
        def body(i, carry):
            off = pl.multiple_of(base + i * _CHUNK, _CHUNK)
            pltpu.sync_copy(idx_hbm.at[pl.ds(off, _CHUNK)], idx_v)
            pltpu.async_copy(tab_hbm.at[idx_v], rows_v, sem).wait()
            pltpu.sync_copy(rows_v, out_hbm.at[pl.ds(off, _CHUNK)])
            return carry
        lax.fori_loop(0, _NCHUNK, body, 0)

    do_table(idx_item_hbm, item_tab_hbm, out_item_hbm)
    do_table(idx_cate_hbm, cate_tab_hbm, out_cate_hbm)


def kernel(indices_item, indices_cate, item_table, cate_table):
    item_vals, cate_vals = _gather_kernel(
        indices_item.reshape(-1), indices_cate.reshape(-1),
        item_table, cate_table)
    return item_vals, cate_vals


# speedup vs baseline: 6.1495x; 1.2774x over previous
import functools

import jax
import jax.numpy as jnp
from jax import lax
from jax.experimental import pallas as pl
from jax.experimental.pallas import tpu as pltpu
from jax.experimental.pallas import tpu_sc as plsc

BATCH = 4096
HIST = 50
DIM = 128
TOTAL = BATCH * HIST

_info = plsc.get_sparse_core_info()
_NC, _NS = _info.num_cores, _info.num_subcores
_NW = _NC * _NS
_PER_W = TOTAL // _NW
_CHUNK = 128
_NCHUNK = _PER_W // _CHUNK
_NPAIR = _NCHUNK // 2

_mesh = plsc.VectorSubcoreMesh(core_axis_name="c", subcore_axis_name="s")


@functools.partial(
    pl.kernel,
    mesh=_mesh,
    out_type=(
        jax.ShapeDtypeStruct((TOTAL, DIM), jnp.float32),
        jax.ShapeDtypeStruct((TOTAL, DIM), jnp.float32),
    ),
    scratch_types=[
        pltpu.VMEM((_PER_W,), jnp.int32),
        pltpu.VMEM((_CHUNK, DIM), jnp.float32),
        pltpu.VMEM((_CHUNK, DIM), jnp.float32),
        pltpu.SemaphoreType.DMA,
        pltpu.SemaphoreType.DMA,
        pltpu.SemaphoreType.DMA,
        pltpu.SemaphoreType.DMA,
    ],
)
def _gather_kernel(idx_item_hbm, idx_cate_hbm, item_tab_hbm, cate_tab_hbm,
                   out_item_hbm, out_cate_hbm,
                   idx_v, rows0, rows1, gsem0, gsem1, ssem0, ssem1):
    wid = lax.axis_index("s") * _NC + lax.axis_index("c")
    base = wid * _PER_W
    rows = (rows0, rows1)
    gsem = (gsem0, gsem1)
    ssem = (ssem0, ssem1)

    def do_table(idx_hbm, tab_hbm, out_hbm):
        pltpu.sync_copy(idx_hbm.at[pl.ds(base, _PER_W)], idx_v)

        def gather_desc(i, b):
            off = pl.multiple_of(i * _CHUNK, _CHUNK)
            return pltpu.make_async_copy(
                tab_hbm.at[idx_v.at[pl.ds(off, _CHUNK)]], rows[b], gsem[b])

        def store_desc(i, b):
            off = pl.multiple_of(base + i * _CHUNK, _CHUNK)
            return pltpu.make_async_copy(
                rows[b], out_hbm.at[pl.ds(off, _CHUNK)], ssem[b])

        gather_desc(0, 0).start()

        def body(p, carry):
            i0 = p * 2
            gather_desc(i0, 0).wait()
            store_desc(i0, 0).start()
            @pl.when(p > 0)
            def _():
                store_desc(i0 - 1, 1).wait()
            gather_desc(i0 + 1, 1).start()
            gather_desc(i0 + 1, 1).wait()
            store_desc(i0 + 1, 1).start()
            store_desc(i0, 0).wait()
            @pl.when(p + 1 < _NPAIR)
            def _():
                gather_desc(i0 + 2, 0).start()
            return carry

        lax.fori_loop(0, _NPAIR, body, 0)
        store_desc(_NCHUNK - 1, 1).wait()

    do_table(idx_item_hbm, item_tab_hbm, out_item_hbm)
    do_table(idx_cate_hbm, cate_tab_hbm, out_cate_hbm)


def kernel(indices_item, indices_cate, item_table, cate_table):
    item_vals, cate_vals = _gather_kernel(
        indices_item.reshape(-1), indices_cate.reshape(-1),
        item_table, cate_table)
    return item_vals, cate_vals


# speedup vs baseline: 6.8172x; 1.1086x over previous
import functools

import jax
import jax.numpy as jnp
from jax import lax
from jax.experimental import pallas as pl
from jax.experimental.pallas import tpu as pltpu
from jax.experimental.pallas import tpu_sc as plsc

BATCH = 4096
HIST = 50
DIM = 128
TOTAL = BATCH * HIST

_info = plsc.get_sparse_core_info()
_NC, _NS = _info.num_cores, _info.num_subcores
_NW = _NC * _NS
_PER_W = TOTAL // _NW
_CHUNK = 128
_NCHUNK = _PER_W // _CHUNK
_NBUF = 5
_NGROUP = _NCHUNK // _NBUF

_mesh = plsc.VectorSubcoreMesh(core_axis_name="c", subcore_axis_name="s")


@functools.partial(
    pl.kernel,
    mesh=_mesh,
    out_type=(
        jax.ShapeDtypeStruct((TOTAL, DIM), jnp.float32),
        jax.ShapeDtypeStruct((TOTAL, DIM), jnp.float32),
    ),
    scratch_types=(
        [pltpu.VMEM((_PER_W,), jnp.int32)]
        + [pltpu.VMEM((_CHUNK, DIM), jnp.float32) for _ in range(_NBUF)]
        + [pltpu.SemaphoreType.DMA for _ in range(2 * _NBUF)]
    ),
)
def _gather_kernel(idx_item_hbm, idx_cate_hbm, item_tab_hbm, cate_tab_hbm,
                   out_item_hbm, out_cate_hbm, idx_v, *bufs_and_sems):
    rows = bufs_and_sems[:_NBUF]
    gsem = bufs_and_sems[_NBUF:2 * _NBUF]
    ssem = bufs_and_sems[2 * _NBUF:]
    wid = lax.axis_index("s") * _NC + lax.axis_index("c")
    base = wid * _PER_W

    def do_table(idx_hbm, tab_hbm, out_hbm):
        pltpu.sync_copy(idx_hbm.at[pl.ds(base, _PER_W)], idx_v)

        def gather_desc(i, b):
            off = pl.multiple_of(i * _CHUNK, _CHUNK)
            return pltpu.make_async_copy(
                tab_hbm.at[idx_v.at[pl.ds(off, _CHUNK)]], rows[b], gsem[b])

        def store_desc(i, b):
            off = pl.multiple_of(base + i * _CHUNK, _CHUNK)
            return pltpu.make_async_copy(
                rows[b], out_hbm.at[pl.ds(off, _CHUNK)], ssem[b])

        for b in range(_NBUF - 1):
            gather_desc(b, b).start()

        def body(q, carry):
            for b in range(_NBUF):
                i = q * _NBUF + b
                gather_desc(i, b).wait()
                store_desc(i, b).start()
                nb = (b + _NBUF - 1) % _NBUF
                if b == 0:
                    @pl.when(q > 0)
                    def _():
                        store_desc(i - 1, nb).wait()
                        gather_desc(i + _NBUF - 1, nb).start()
                    @pl.when(q == 0)
                    def _():
                        gather_desc(i + _NBUF - 1, nb).start()
                else:
                    store_desc(i - 1, nb).wait()
                    @pl.when(i + _NBUF - 1 < _NCHUNK)
                    def _():
                        gather_desc(i + _NBUF - 1, nb).start()
            return carry

        lax.fori_loop(0, _NGROUP, body, 0)
        store_desc(_NCHUNK - 1, (_NCHUNK - 1) % _NBUF).wait()

    do_table(idx_item_hbm, item_tab_hbm, out_item_hbm)
    do_table(idx_cate_hbm, cate_tab_hbm, out_cate_hbm)


def kernel(indices_item, indices_cate, item_table, cate_table):
    item_vals, cate_vals = _gather_kernel(
        indices_item.reshape(-1), indices_cate.reshape(-1),
        item_table, cate_table)
    return item_vals, cate_vals


# speedup vs baseline: 10.6440x; 1.5613x over previous
import functools

import jax
import jax.numpy as jnp
from jax import lax
from jax.experimental import pallas as pl
from jax.experimental.pallas import tpu as pltpu
from jax.experimental.pallas import tpu_sc as plsc

BATCH = 4096
HIST = 50
DIM = 128
TOTAL = BATCH * HIST
CATE_VOCAB = 1000

_info = plsc.get_sparse_core_info()
_NC, _NS = _info.num_cores, _info.num_subcores
_NW = _NC * _NS
_PER_W = TOTAL // _NW
_CHUNK = 128
_NCHUNK = _PER_W // _CHUNK
_NBUF = 5
_NGROUP = _NCHUNK // _NBUF

_mesh = plsc.VectorSubcoreMesh(core_axis_name="c", subcore_axis_name="s")


@functools.partial(
    pl.kernel,
    mesh=_mesh,
    out_type=(
        jax.ShapeDtypeStruct((TOTAL, DIM), jnp.float32),
        jax.ShapeDtypeStruct((TOTAL, DIM), jnp.float32),
    ),
    scratch_types=(
        [pltpu.VMEM((_PER_W,), jnp.int32)]
        + [pltpu.VMEM((_CHUNK, DIM), jnp.float32) for _ in range(_NBUF)]
        + [pltpu.VMEM_SHARED((CATE_VOCAB, DIM), jnp.float32)]
        + [pltpu.SemaphoreType.DMA for _ in range(2 * _NBUF + 1)]
    ),
)
def _gather_kernel(idx_item_hbm, idx_cate_hbm, item_tab_hbm, cate_tab_hbm,
                   out_item_hbm, out_cate_hbm, idx_v, *bufs_and_sems):
    rows = bufs_and_sems[:_NBUF]
    cate_spmem = bufs_and_sems[_NBUF]
    gsem = bufs_and_sems[_NBUF + 1:2 * _NBUF + 1]
    ssem = bufs_and_sems[2 * _NBUF + 1:3 * _NBUF + 1]
    stsem = bufs_and_sems[3 * _NBUF + 1]
    sid = lax.axis_index("s")
    wid = sid * _NC + lax.axis_index("c")
    base = wid * _PER_W

    def do_table(idx_hbm, tab_hbm, out_hbm):
        pltpu.sync_copy(idx_hbm.at[pl.ds(base, _PER_W)], idx_v)

        def gather_desc(i, b):
            off = pl.multiple_of(i * _CHUNK, _CHUNK)
            return pltpu.make_async_copy(
                tab_hbm.at[idx_v.at[pl.ds(off, _CHUNK)]], rows[b], gsem[b])

        def store_desc(i, b):
            off = pl.multiple_of(base + i * _CHUNK, _CHUNK)
            return pltpu.make_async_copy(
                rows[b], out_hbm.at[pl.ds(off, _CHUNK)], ssem[b])

        for b in range(_NBUF - 1):
            gather_desc(b, b).start()

        def body(q, carry):
            for b in range(_NBUF):
                i = q * _NBUF + b
                gather_desc(i, b).wait()
                store_desc(i, b).start()
                nb = (b + _NBUF - 1) % _NBUF
                if b == 0:
                    @pl.when(q > 0)
                    def _():
                        store_desc(i - 1, nb).wait()
                        gather_desc(i + _NBUF - 1, nb).start()
                    @pl.when(q == 0)
                    def _():
                        gather_desc(i + _NBUF - 1, nb).start()
                else:
                    store_desc(i - 1, nb).wait()
                    @pl.when(i + _NBUF - 1 < _NCHUNK)
                    def _():
                        gather_desc(i + _NBUF - 1, nb).start()
            return carry

        lax.fori_loop(0, _NGROUP, body, 0)
        store_desc(_NCHUNK - 1, (_NCHUNK - 1) % _NBUF).wait()

    stage = pltpu.make_async_copy(cate_tab_hbm, cate_spmem, stsem)

    @pl.when(sid == 0)
    def _():
        stage.start()

    do_table(idx_item_hbm, item_tab_hbm, out_item_hbm)

    @pl.when(sid == 0)
    def _():
        stage.wait()

    plsc.subcore_barrier()
    do_table(idx_cate_hbm, cate_spmem, out_cate_hbm)


def kernel(indices_item, indices_cate, item_table, cate_table):
    item_vals, cate_vals = _gather_kernel(
        indices_item.reshape(-1), indices_cate.reshape(-1),
        item_table, cate_table)
    return item_vals, cate_vals
